# Initial kernel scaffold; baseline (speedup 1.0000x reference)
import jax
import jax.numpy as jnp
from jax.experimental import pallas as pl


def kernel(idx, table):
    raise NotImplementedError("write your pallas kernel here")


# speedup vs baseline: 1.0302x; 1.0302x over previous
import functools

import jax
import jax.numpy as jnp
from jax import lax
from jax.experimental import pallas as pl
from jax.experimental.pallas import tpu as pltpu
from jax.experimental.pallas import tpu_sc as plsc

NC = 2
NS = 16
NW = NC * NS
C = 64


def _sc_gather(idx_grp, table, total, vocab):
    chunks = idx_grp.shape[1]
    per_w = chunks * C
    mesh = plsc.VectorSubcoreMesh(core_axis_name="c", subcore_axis_name="s")

    @functools.partial(
        pl.kernel,
        mesh=mesh,
        out_type=jax.ShapeDtypeStruct((total, vocab), jnp.float32),
        compiler_params=pltpu.CompilerParams(use_tc_tiling_on_sc=False),
        scratch_types=[
            pltpu.VMEM((chunks, C), jnp.int32),
            pltpu.VMEM((C, vocab), jnp.float32),
            pltpu.VMEM((C, vocab), jnp.float32),
            pltpu.SemaphoreType.DMA,
            pltpu.SemaphoreType.DMA,
        ],
    )
    def k(idx_hbm, table_hbm, out_hbm, idx_v, rows0, rows1, sem0, sem1):
        wid = lax.axis_index("s") * NC + lax.axis_index("c")
        base = wid * per_w
        pltpu.sync_copy(idx_hbm.at[wid], idx_v)

        pltpu.async_copy(table_hbm.at[idx_v.at[0]], rows0, sem0)

        def chunk(i, carry):
            for b, (rows, sem, osem, orows) in enumerate(
                ((rows0, sem0, sem1, rows1), (rows1, sem1, sem0, rows0))):
                j = i + b
                @pl.when(j + 1 < chunks)
                def _():
                    pltpu.async_copy(
                        table_hbm.at[idx_v.at[j + 1]], orows, osem)
                pltpu.make_async_copy(
                    table_hbm.at[idx_v.at[j]], rows, sem).wait()
                pltpu.sync_copy(rows, out_hbm.at[pl.ds(base + j * C, C)])
            return carry

        lax.fori_loop(0, chunks // 2, lambda i, c: chunk(2 * i, c), 0)
        if chunks % 2 == 1:
            j = chunks - 1
            pltpu.make_async_copy(
                table_hbm.at[idx_v.at[j]], rows0, sem0).wait()
            pltpu.sync_copy(rows0, out_hbm.at[pl.ds(base + j * C, C)])

    return k(idx_grp, table)


def kernel(idx, table):
    b, t = idx.shape
    vocab = table.shape[1]
    total = b * t
    per_w = total // NW
    chunks = per_w // C
    idx_grp = idx.astype(jnp.int32).reshape(NW, chunks, C)
    out = _sc_gather(idx_grp, table, total, vocab)
    return out.reshape(b, t, vocab)


# speedup vs baseline: 1.0335x; 1.0032x over previous
import functools

import jax
import jax.numpy as jnp
from jax import lax
from jax.experimental import pallas as pl
from jax.experimental.pallas import tpu as pltpu
from jax.experimental.pallas import tpu_sc as plsc

NC = 2
NS = 16
NW = NC * NS
C = 64


def _sc_gather(idx_grp, table, total, vocab):
    chunks = idx_grp.shape[1]
    per_w = chunks * C
    mesh = plsc.VectorSubcoreMesh(core_axis_name="c", subcore_axis_name="s")

    @functools.partial(
        pl.kernel,
        mesh=mesh,
        out_type=jax.ShapeDtypeStruct((total, vocab), jnp.float32),
        compiler_params=pltpu.CompilerParams(use_tc_tiling_on_sc=False),
        scratch_types=[
            pltpu.VMEM((chunks, C), jnp.int32),
            pltpu.VMEM((C, vocab), jnp.float32),
            pltpu.VMEM((C, vocab), jnp.float32),
            pltpu.SemaphoreType.DMA,
            pltpu.SemaphoreType.DMA,
            pltpu.SemaphoreType.DMA,
            pltpu.SemaphoreType.DMA,
        ],
    )
    def k(idx_hbm, table_hbm, out_hbm, idx_v, rows0, rows1, g0, g1, s0, s1):
        wid = lax.axis_index("s") * NC + lax.axis_index("c")
        base = wid * per_w
        pltpu.sync_copy(idx_hbm.at[wid], idx_v)

        rows = (rows0, rows1)
        gsem = (g0, g1)
        ssem = (s0, s1)

        def gather_start(j, b):
            pltpu.async_copy(table_hbm.at[idx_v.at[j]], rows[b], gsem[b])

        def gather_wait(j, b):
            pltpu.make_async_copy(
                table_hbm.at[idx_v.at[j]], rows[b], gsem[b]).wait()

        def scatter_start(j, b):
            pltpu.async_copy(
                rows[b], out_hbm.at[pl.ds(base + j * C, C)], ssem[b])

        def scatter_wait(b):
            pltpu.make_async_copy(
                rows[b], out_hbm.at[pl.ds(base, C)], ssem[b]).wait()

        gather_start(0, 0)

        def pair(i, carry):
            for b in (0, 1):
                j = 2 * i + b
                ob = 1 - b

                @pl.when(j + 1 < chunks)
                def _():
                    @pl.when(j >= 1)
                    def _():
                        scatter_wait(ob)
                    gather_start(j + 1, ob)

                gather_wait(j, b)
                scatter_start(j, b)
            return carry

        lax.fori_loop(0, chunks // 2, pair, 0)
        if chunks % 2 == 1:
            j = chunks - 1
            scatter_wait(1)
            gather_wait(j, 0)
            scatter_start(j, 0)
            scatter_wait(0)
        else:
            scatter_wait(0)
            scatter_wait(1)

    return k(idx_grp, table)


def kernel(idx, table):
    b, t = idx.shape
    vocab = table.shape[1]
    total = b * t
    per_w = total // NW
    chunks = per_w // C
    idx_grp = idx.astype(jnp.int32).reshape(NW, chunks, C)
    out = _sc_gather(idx_grp, table, total, vocab)
    return out.reshape(b, t, vocab)


# speedup vs baseline: 1.0364x; 1.0028x over previous
import functools

import jax
import jax.numpy as jnp
from jax import lax
from jax.experimental import pallas as pl
from jax.experimental.pallas import tpu as pltpu
from jax.experimental.pallas import tpu_sc as plsc

NC = 2
NS = 16
NW = NC * NS


def _sc_gather(idx_grp, table, b_total, t_len, vocab):
    slabs = idx_grp.shape[1]
    mesh = plsc.VectorSubcoreMesh(core_axis_name="c", subcore_axis_name="s")

    @functools.partial(
        pl.kernel,
        mesh=mesh,
        out_type=jax.ShapeDtypeStruct((b_total, t_len, vocab), jnp.float32),
        compiler_params=pltpu.CompilerParams(use_tc_tiling_on_sc=False),
        scratch_types=[
            pltpu.VMEM((slabs, t_len), jnp.int32),
            pltpu.VMEM((t_len, vocab), jnp.float32),
            pltpu.VMEM((t_len, vocab), jnp.float32),
            pltpu.SemaphoreType.DMA,
            pltpu.SemaphoreType.DMA,
            pltpu.SemaphoreType.DMA,
            pltpu.SemaphoreType.DMA,
        ],
    )
    def k(idx_hbm, table_hbm, out_hbm, idx_v, rows0, rows1, g0, g1, s0, s1):
        wid = lax.axis_index("s") * NC + lax.axis_index("c")
        base = wid * slabs
        pltpu.sync_copy(idx_hbm.at[wid], idx_v)

        rows = (rows0, rows1)
        gsem = (g0, g1)
        ssem = (s0, s1)

        def gather_start(j, b):
            pltpu.async_copy(table_hbm.at[idx_v.at[j]], rows[b], gsem[b])

        def gather_wait(j, b):
            pltpu.make_async_copy(
                table_hbm.at[idx_v.at[j]], rows[b], gsem[b]).wait()

        def scatter_start(j, b):
            pltpu.async_copy(rows[b], out_hbm.at[base + j], ssem[b])

        def scatter_wait(b):
            pltpu.make_async_copy(rows[b], out_hbm.at[base], ssem[b]).wait()

        gather_start(0, 0)

        def pair(i, carry):
            for b in (0, 1):
                j = 2 * i + b
                ob = 1 - b

                @pl.when(j + 1 < slabs)
                def _():
                    @pl.when(j >= 1)
                    def _():
                        scatter_wait(ob)
                    gather_start(j + 1, ob)

                gather_wait(j, b)
                scatter_start(j, b)
            return carry

        lax.fori_loop(0, slabs // 2, pair, 0)
        if slabs % 2 == 1:
            j = slabs - 1
            scatter_wait(1)
            gather_wait(j, 0)
            scatter_start(j, 0)
            scatter_wait(0)
        else:
            scatter_wait(0)
            scatter_wait(1)

    return k(idx_grp, table)


def kernel(idx, table):
    b, t = idx.shape
    vocab = table.shape[1]
    idx_grp = idx.astype(jnp.int32).reshape(NW, b // NW, t)
    return _sc_gather(idx_grp, table, b, t, vocab)


# speedup vs baseline: 1.4114x; 1.3618x over previous
import functools

import jax
import jax.numpy as jnp
from jax import lax
from jax.experimental import pallas as pl
from jax.experimental.pallas import tpu as pltpu
from jax.experimental.pallas import tpu_sc as plsc

NC = 2
NS = 16
NW = NC * NS
BW = 1024 // NW


def _sc_gather(idx_grp, table_pad, b_total, t_len, vpad):
    mesh = plsc.VectorSubcoreMesh(core_axis_name="c", subcore_axis_name="s")

    @functools.partial(
        pl.kernel,
        mesh=mesh,
        out_type=jax.ShapeDtypeStruct((t_len, b_total, vpad), jnp.float32),
        scratch_types=[
            pltpu.VMEM((t_len, BW), jnp.int32),
            pltpu.VMEM((BW, vpad), jnp.float32),
            pltpu.VMEM((BW, vpad), jnp.float32),
            pltpu.SemaphoreType.DMA,
            pltpu.SemaphoreType.DMA,
            pltpu.SemaphoreType.DMA,
            pltpu.SemaphoreType.DMA,
        ],
    )
    def k(idx_hbm, table_hbm, out_hbm, idx_v, rows0, rows1, g0, g1, s0, s1):
        wid = lax.axis_index("s") * NC + lax.axis_index("c")
        base = wid * BW
        pltpu.sync_copy(idx_hbm.at[wid], idx_v)

        rows = (rows0, rows1)
        gsem = (g0, g1)
        ssem = (s0, s1)

        def gather_start(t, b):
            pltpu.async_copy(table_hbm.at[idx_v.at[t]], rows[b], gsem[b])

        def gather_wait(t, b):
            pltpu.make_async_copy(
                table_hbm.at[idx_v.at[t]], rows[b], gsem[b]).wait()

        def scatter_start(t, b):
            pltpu.async_copy(
                rows[b], out_hbm.at[t].at[pl.ds(base, BW), :], ssem[b])

        def scatter_wait(b):
            pltpu.make_async_copy(
                rows[b], out_hbm.at[0].at[pl.ds(base, BW), :], ssem[b]).wait()

        gather_start(0, 0)

        def pair(i, carry):
            for b in (0, 1):
                t = 2 * i + b
                ob = 1 - b

                @pl.when(t + 1 < t_len)
                def _():
                    @pl.when(t >= 1)
                    def _():
                        scatter_wait(ob)
                    gather_start(t + 1, ob)

                gather_wait(t, b)
                scatter_start(t, b)
            return carry

        lax.fori_loop(0, t_len // 2, pair, 0)
        if t_len % 2 == 1:
            t = t_len - 1
            scatter_wait(1)
            gather_wait(t, 0)
            scatter_start(t, 0)
            scatter_wait(0)
        else:
            scatter_wait(0)
            scatter_wait(1)

    return k(idx_grp, table_pad)


def _tc_transpose(stage1, t_len, b_total, vocab, vpad):
    def body(x_ref, o_ref):
        o_ref[0] = jnp.transpose(x_ref[0], (1, 0))[:vocab, :]

    return pl.pallas_call(
        body,
        grid=(t_len, b_total // 128),
        in_specs=[pl.BlockSpec((1, 128, vpad), lambda t, k: (t, k, 0))],
        out_specs=pl.BlockSpec((1, vocab, 128), lambda t, k: (t, 0, k)),
        out_shape=jax.ShapeDtypeStruct((t_len, vocab, b_total), jnp.float32),
    )(stage1)


def kernel(idx, table):
    b_total, t_len = idx.shape
    vocab = table.shape[1]
    vpad = (vocab + 127) // 128 * 128
    table_pad = jnp.pad(table, ((0, 0), (0, vpad - vocab)))
    idx_grp = idx.astype(jnp.int32).reshape(NW, BW, t_len).transpose(0, 2, 1)
    stage1 = _sc_gather(idx_grp, table_pad, b_total, t_len, vpad)
    tvb = _tc_transpose(stage1, t_len, b_total, vocab, vpad)
    return jnp.transpose(tvb, (2, 0, 1))


# speedup vs baseline: 2.2354x; 1.5839x over previous
import functools

import jax
import jax.numpy as jnp
from jax import lax
from jax.experimental import pallas as pl
from jax.experimental.pallas import tpu as pltpu
from jax.experimental.pallas import tpu_sc as plsc

NC = 2
NS = 16
NW = NC * NS
BW = 1024 // NW


def _sc_gather(idx_grp, table_pad, b_total, t_len, vpad):
    mesh = plsc.VectorSubcoreMesh(core_axis_name="c", subcore_axis_name="s")

    @functools.partial(
        pl.kernel,
        mesh=mesh,
        out_type=jax.ShapeDtypeStruct((t_len, b_total, vpad), jnp.float32),
        scratch_types=[
            pltpu.VMEM((t_len, BW), jnp.int32),
            pltpu.VMEM((BW, vpad), jnp.float32),
            pltpu.VMEM((BW, vpad), jnp.float32),
            pltpu.SemaphoreType.DMA,
            pltpu.SemaphoreType.DMA,
            pltpu.SemaphoreType.DMA,
            pltpu.SemaphoreType.DMA,
        ],
    )
    def k(idx_hbm, table_hbm, out_hbm, idx_v, rows0, rows1, g0, g1, s0, s1):
        wid = lax.axis_index("s") * NC + lax.axis_index("c")
        base = wid * BW
        pltpu.sync_copy(idx_hbm.at[wid], idx_v)

        rows = (rows0, rows1)
        gsem = (g0, g1)
        ssem = (s0, s1)

        def gather_start(t, b):
            pltpu.async_copy(table_hbm.at[idx_v.at[t]], rows[b], gsem[b])

        def gather_wait(t, b):
            pltpu.make_async_copy(
                table_hbm.at[idx_v.at[t]], rows[b], gsem[b]).wait()

        def scatter_start(t, b):
            pltpu.async_copy(
                rows[b], out_hbm.at[t].at[pl.ds(base, BW), :], ssem[b])

        def scatter_wait(b):
            pltpu.make_async_copy(
                rows[b], out_hbm.at[0].at[pl.ds(base, BW), :], ssem[b]).wait()

        gather_start(0, 0)

        def pair(i, carry):
            for b in (0, 1):
                t = 2 * i + b
                ob = 1 - b

                @pl.when(t + 1 < t_len)
                def _():
                    @pl.when(t >= 1)
                    def _():
                        scatter_wait(ob)
                    gather_start(t + 1, ob)

                gather_wait(t, b)
                scatter_start(t, b)
            return carry

        lax.fori_loop(0, t_len // 2, pair, 0)
        if t_len % 2 == 1:
            t = t_len - 1
            scatter_wait(1)
            gather_wait(t, 0)
            scatter_start(t, 0)
            scatter_wait(0)
        else:
            scatter_wait(0)
            scatter_wait(1)

    return k(idx_grp, table_pad)


def _tc_transpose(stage1, t_len, b_total, vocab, vpad):
    def body(x_ref, o_ref):
        o_ref[0] = jnp.transpose(x_ref[0], (1, 0))[:vocab, :]

    return pl.pallas_call(
        body,
        grid=(t_len,),
        in_specs=[pl.BlockSpec((1, b_total, vpad), lambda t: (t, 0, 0))],
        out_specs=pl.BlockSpec((1, vocab, b_total), lambda t: (t, 0, 0)),
        out_shape=jax.ShapeDtypeStruct((t_len, vocab, b_total), jnp.float32),
    )(stage1)


def kernel(idx, table):
    b_total, t_len = idx.shape
    vocab = table.shape[1]
    vpad = (vocab + 127) // 128 * 128
    table_pad = jnp.pad(table, ((0, 0), (0, vpad - vocab)))
    idx_grp = idx.astype(jnp.int32).reshape(NW, BW, t_len).transpose(0, 2, 1)
    stage1 = _sc_gather(idx_grp, table_pad, b_total, t_len, vpad)
    tvb = _tc_transpose(stage1, t_len, b_total, vocab, vpad)
    return jnp.transpose(tvb, (2, 0, 1))
